# baseline (speedup 1.0000x reference)
import functools

import jax
import jax.numpy as jnp
from jax import lax
from jax.experimental import pallas as pl
from jax.experimental.pallas import tpu as pltpu
from jax.experimental.pallas import tpu_sc as plsc

B = 16384
F = 16
NC = 2
NS = 16
NW = NC * NS
BPW = B // NW
CHUNK = 128
NCHUNK = BPW // CHUNK
RPK = 8


def _mf_kernel(uj_hbm, ij_hbm, ur_hbm, ir_hbm, user_hbm, item_hbm, out_hbm,
               u_idx, i_idx, u_rem, i_rem, u_rows0, u_rows1, i_rows0, i_rows1,
               out_v, sem_u, sem_i):
    wid = lax.axis_index("s") * NC + lax.axis_index("c")

    pltpu.sync_copy(uj_hbm.at[wid], u_idx)
    pltpu.sync_copy(ij_hbm.at[wid], i_idx)
    pltpu.sync_copy(ur_hbm.at[wid], u_rem)
    pltpu.sync_copy(ir_hbm.at[wid], i_rem)

    u_bufs = (u_rows0, u_rows1)
    i_bufs = (i_rows0, i_rows1)

    def fire(c):
        uc = pltpu.async_copy(user_hbm.at[u_idx.at[c]], u_bufs[c % 2], sem_u)
        ic = pltpu.async_copy(item_hbm.at[i_idx.at[c]], i_bufs[c % 2], sem_i)
        return uc, ic

    lanes = lax.iota(jnp.int32, F)
    pend = fire(0)
    for c in range(NCHUNK):
        nxt_pend = fire(c + 1) if c + 1 < NCHUNK else None
        pend[0].wait()
        pend[1].wait()
        ub = u_bufs[c % 2]
        ib = i_bufs[c % 2]

        def body(g, carry, c=c, ub=ub, ib=ib):
            rows16 = g * F + lanes
            ucol = u_rem[c, pl.ds(g * F, F)] * F
            icol = i_rem[c, pl.ds(g * F, F)] * F
            acc0 = (plsc.load_gather(ub, [rows16, ucol]) *
                    plsc.load_gather(ib, [rows16, icol]))
            acc1 = (plsc.load_gather(ub, [rows16, ucol + 1]) *
                    plsc.load_gather(ib, [rows16, icol + 1]))
            for f in range(2, F, 2):
                acc0 = acc0 + (plsc.load_gather(ub, [rows16, ucol + f]) *
                               plsc.load_gather(ib, [rows16, icol + f]))
                acc1 = acc1 + (plsc.load_gather(ub, [rows16, ucol + f + 1]) *
                               plsc.load_gather(ib, [rows16, icol + f + 1]))
            out_v[c, pl.ds(g * F, F)] = acc0 + acc1
            return carry

        lax.fori_loop(0, CHUNK // F, body, 0)
        pend = nxt_pend

    pltpu.sync_copy(out_v, out_hbm.at[wid])


@jax.jit
def kernel(u, i, user_emb, item_emb):
    u32 = u.astype(jnp.int32)
    i32 = i.astype(jnp.int32)
    uj = (u32 // RPK).reshape(NW, NCHUNK, CHUNK)
    ij = (i32 // RPK).reshape(NW, NCHUNK, CHUNK)
    ur = (u32 % RPK).reshape(NW, NCHUNK, CHUNK)
    ir = (i32 % RPK).reshape(NW, NCHUNK, CHUNK)
    user_p = user_emb.reshape(1000000 // RPK, RPK * F)
    item_p = item_emb.reshape(1000000 // RPK, RPK * F)

    mesh = plsc.VectorSubcoreMesh(core_axis_name="c", subcore_axis_name="s")
    k = functools.partial(
        pl.kernel,
        out_type=jax.ShapeDtypeStruct((NW, NCHUNK, CHUNK), jnp.float32),
        mesh=mesh,
        compiler_params=pltpu.CompilerParams(
            needs_layout_passes=False, use_tc_tiling_on_sc=True),
        scratch_types=[
            pltpu.VMEM((NCHUNK, CHUNK), jnp.int32),
            pltpu.VMEM((NCHUNK, CHUNK), jnp.int32),
            pltpu.VMEM((NCHUNK, CHUNK), jnp.int32),
            pltpu.VMEM((NCHUNK, CHUNK), jnp.int32),
            pltpu.VMEM((CHUNK, RPK * F), jnp.float32),
            pltpu.VMEM((CHUNK, RPK * F), jnp.float32),
            pltpu.VMEM((CHUNK, RPK * F), jnp.float32),
            pltpu.VMEM((CHUNK, RPK * F), jnp.float32),
            pltpu.VMEM((NCHUNK, CHUNK), jnp.float32),
            pltpu.SemaphoreType.DMA,
            pltpu.SemaphoreType.DMA,
        ],
    )(_mf_kernel)
    out = k(uj, ij, ur, ir, user_p, item_p)
    return out.reshape(B)
